# baseline (speedup 1.0000x reference)
import numpy as np
import jax
import jax.numpy as jnp
from jax.experimental import pallas as pl
from jax.experimental.pallas import tpu as pltpu
from jax.experimental.pallas import tpu_sc as plsc

_B = 1024
_MN = 80
_MC = 20
_M = _MN + _MC
_S = _M + 1
_SP = 104
_D = 64
_H = 8
_DH = _D // _H
_NB = 3
_KP = 256
_FFN = 128
_CARD = 10000
_SEED = 0
_BB = 8
_GW = 128


def _np_mask(block_seed):
    rng = np.random.default_rng(block_seed)
    m = np.zeros((_S, _S), dtype=bool)
    np.fill_diagonal(m, True)
    m[_M, :] = True
    m[:, _M] = True
    ii = rng.integers(0, _M, size=_KP)
    jj = rng.integers(0, _M, size=_KP)
    m[ii, jj] = True
    m[jj, ii] = True
    return m


def _np_mask_bias():
    bias = np.full((_NB, _SP, _SP), -1e9, dtype=np.float32)
    for i in range(_NB):
        m = _np_mask(_SEED + i)
        bias[i, :_S, :_S] = np.where(m, 0.0, -1e9).astype(np.float32)
    return bias


_MASK_BIAS = _np_mask_bias()
_SCALE = 1.0 / np.sqrt(_DH).astype(np.float32)


def _sc_gather(table, flat_idx):
    n = flat_idx.shape[0]
    idx2 = flat_idx.reshape(1, n)
    mesh = plsc.VectorSubcoreMesh(core_axis_name="core", subcore_axis_name="subcore")

    @pl.kernel(out_type=jax.ShapeDtypeStruct((n, table.shape[1]), table.dtype),
               mesh=mesh)
    def gk(tab_hbm, i_hbm, o_hbm):
        def body(i_vmem, o_vmem):
            pltpu.sync_copy(tab_hbm.at[i_vmem.at[0]], o_vmem)

        pltpu.emit_pipeline(
            body,
            grid=(n // _GW,),
            in_specs=[pl.BlockSpec((1, _GW), index_map=lambda i: (0, i))],
            out_specs=[pl.BlockSpec((_GW, table.shape[1]),
                                    index_map=lambda i: (i, 0))],
            core_axis_name=("core", "subcore"),
            dimension_semantics=(pltpu.PARALLEL,),
        )(i_hbm, o_hbm)

    return gk(table, idx2)


def _ln(x, s, b, eps=1e-5):
    mu = x.mean(-1, keepdims=True)
    var = ((x - mu) ** 2).mean(-1, keepdims=True)
    return (x - mu) / jnp.sqrt(var + eps) * s + b


def _tc_body(xnum_ref, cat_ref, numw_ref, numb_ref, catb_ref, cls_ref,
             wqkv_ref, bqkv_ref, wo_ref, bo_ref, ln1s_ref, ln1b_ref,
             w1_ref, b1_ref, w2_ref, b2_ref, ln2s_ref, ln2b_ref,
             hlns_ref, hlnb_ref, hw_ref, hb_ref, bias_ref,
             y_ref, attn_ref):
    xn = xnum_ref[...]
    num_tok = numw_ref[...][None] * xn[:, :, None] + numb_ref[...][None]
    cat_tok = cat_ref[...][:, :, :_D] + catb_ref[...][None]
    cls_tok = jnp.broadcast_to(cls_ref[...][None], (_BB, 1, _D))
    pad = jnp.zeros((_BB, _SP - _S, _D), jnp.float32)
    x = jnp.concatenate([num_tok, cat_tok, cls_tok, pad], axis=1)

    for i in range(_NB):
        xr = _ln(x, ln1s_ref[i], ln1b_ref[i])
        x2 = xr.reshape(_BB * _SP, _D)
        qkv = x2 @ wqkv_ref[i] + bqkv_ref[i]
        q = qkv[:, :_D].reshape(_BB, _SP, _D)
        k = qkv[:, _D:2 * _D].reshape(_BB, _SP, _D)
        v = qkv[:, 2 * _D:].reshape(_BB, _SP, _D)
        bias = bias_ref[i][None]
        ps = []
        for h in range(_H):
            qh = q[:, :, h * _DH:(h + 1) * _DH]
            kh = k[:, :, h * _DH:(h + 1) * _DH]
            lg = jax.lax.dot_general(
                qh, kh, (((2,), (2,)), ((0,), (0,)))) * _SCALE + bias
            mx = lg.max(-1, keepdims=True)
            e = jnp.exp(lg - mx)
            p = e / e.sum(-1, keepdims=True)
            ps.append(p)
        os_ = []
        for h in range(_H):
            vh = v[:, :, h * _DH:(h + 1) * _DH]
            oh = jax.lax.dot_general(
                ps[h], vh, (((2,), (1,)), ((0,), (0,))))
            os_.append(oh)
        o = jnp.concatenate(os_, axis=-1).reshape(_BB * _SP, _D)
        o = o @ wo_ref[i] + bo_ref[i]
        x = x + o.reshape(_BB, _SP, _D)
        if i == _NB - 1:
            pm = ps[0]
            for h in range(1, _H):
                pm = pm + ps[h]
            attn_ref[...] = pm * (1.0 / _H)
        xr2 = _ln(x, ln2s_ref[i], ln2b_ref[i])
        hmid = xr2.reshape(_BB * _SP, _D) @ w1_ref[i] + b1_ref[i]
        a = hmid[:, :_FFN]
        g = hmid[:, _FFN:]
        ffn = (a * jnp.maximum(g, 0.0)) @ w2_ref[i] + b2_ref[i]
        x = x + ffn.reshape(_BB, _SP, _D)

    cls_out = x[:, _M, :]
    yv = _ln(cls_out, hlns_ref[...], hlnb_ref[...])
    yv = jnp.maximum(yv, 0.0)
    y_ref[...] = (yv * hw_ref[...]).sum(-1, keepdims=True) + hb_ref[...]


def _full(shape):
    nd = len(shape)
    return pl.BlockSpec(shape, lambda g: (0,) * nd)


def kernel(x_num, x_cat, params):
    p = params
    b = x_num.shape[0]
    offs = (jnp.arange(_MC, dtype=x_cat.dtype) * _CARD)[None]
    flat_idx = (x_cat + offs).reshape(-1)
    tabp = jnp.pad(p['cat_emb'], ((0, 0), (0, 2 * _D - _D)))
    cat_tok = _sc_gather(tabp, flat_idx).reshape(b, _MC, 2 * _D)

    blocks = p['blocks']
    wqkv = jnp.stack([jnp.concatenate([bl['wq'], bl['wk'], bl['wv']], axis=1)
                      for bl in blocks])
    bqkv = jnp.stack([jnp.concatenate([bl['bq'], bl['bk'], bl['bv']])
                      for bl in blocks]).reshape(_NB, 1, 3 * _D)
    wo = jnp.stack([bl['wo'] for bl in blocks])
    bo = jnp.stack([bl['bo'] for bl in blocks]).reshape(_NB, 1, _D)
    ln1s = jnp.stack([bl['ln1_s'] for bl in blocks]).reshape(_NB, 1, _D)
    ln1b = jnp.stack([bl['ln1_b'] for bl in blocks]).reshape(_NB, 1, _D)
    w1 = jnp.stack([bl['w1'] for bl in blocks])
    b1 = jnp.stack([bl['b1'] for bl in blocks]).reshape(_NB, 1, 2 * _FFN)
    w2 = jnp.stack([bl['w2'] for bl in blocks])
    b2 = jnp.stack([bl['b2'] for bl in blocks]).reshape(_NB, 1, _D)
    ln2s = jnp.stack([bl['ln2_s'] for bl in blocks]).reshape(_NB, 1, _D)
    ln2b = jnp.stack([bl['ln2_b'] for bl in blocks]).reshape(_NB, 1, _D)

    cls_w = p['cls'].reshape(1, _D)
    hlns = p['head_ln_s'].reshape(1, _D)
    hlnb = p['head_ln_b'].reshape(1, _D)
    hw = p['head_w'].reshape(1, _D)
    hb = p['head_b'].reshape(1, 1)
    bias = jnp.asarray(_MASK_BIAS)

    grid = (b // _BB,)
    y, attn = pl.pallas_call(
        _tc_body,
        grid=grid,
        in_specs=[
            pl.BlockSpec((_BB, _MN), lambda g: (g, 0)),
            pl.BlockSpec((_BB, _MC, 2 * _D), lambda g: (g, 0, 0)),
            _full((_MN, _D)), _full((_MN, _D)), _full((_MC, _D)),
            _full((1, _D)),
            _full((_NB, _D, 3 * _D)), _full((_NB, 1, 3 * _D)),
            _full((_NB, _D, _D)), _full((_NB, 1, _D)),
            _full((_NB, 1, _D)), _full((_NB, 1, _D)),
            _full((_NB, _D, 2 * _FFN)), _full((_NB, 1, 2 * _FFN)),
            _full((_NB, _FFN, _D)), _full((_NB, 1, _D)),
            _full((_NB, 1, _D)), _full((_NB, 1, _D)),
            _full((1, _D)), _full((1, _D)), _full((1, _D)), _full((1, 1)),
            _full((_NB, _SP, _SP)),
        ],
        out_specs=[
            pl.BlockSpec((_BB, 1), lambda g: (g, 0)),
            pl.BlockSpec((_BB, _SP, _SP), lambda g: (g, 0, 0)),
        ],
        out_shape=[
            jax.ShapeDtypeStruct((b, 1), jnp.float32),
            jax.ShapeDtypeStruct((b, _SP, _SP), jnp.float32),
        ],
    )(x_num, cat_tok, p['num_w'], p['num_b'], p['cat_b'], cls_w,
      wqkv, bqkv, wo, bo, ln1s, ln1b, w1, b1, w2, b2, ln2s, ln2b,
      hlns, hlnb, hw, hb, bias)

    return y, attn[:, :_S, :_S]
